# baseline (speedup 1.0000x reference)
import functools

import jax
import jax.numpy as jnp
from jax import lax
from jax.experimental import pallas as pl
from jax.experimental.pallas import tpu as pltpu
from jax.experimental.pallas import tpu_sc as plsc

N, C, D, GH, GW = 8, 12, 8, 16, 16
H = W = 512
L = 16
NCORES, NSUB = 2, 16
NWORK = NCORES * NSUB
ROWS_PER_W = (N * H) // NWORK
RBLK = 4
NBLK = ROWS_PER_W // RBLK
NCHUNK = W // L
CD = C * D


def _splat_i32(s):
    return lax.broadcast_in_dim(jnp.int32(s) if isinstance(s, int) else s,
                                (L,), ())


def _sc_body(grid_hbm, guide_hbm, out_hbm,
             gridv, gxv, ytab0, ytab1, fytab, guidebuf, outbuf):
    wid = lax.axis_index("s") * NCORES + lax.axis_index("c")
    n = wid // (NWORK // N)
    rowbase = (wid % (NWORK // N)) * ROWS_PER_W

    pltpu.sync_copy(grid_hbm.at[n], gridv)

    def fill_y(ch, carry):
        wv = lax.iota(jnp.int32, L) + _splat_i32(ch * L)
        t = wv * (GH - 1)
        y0 = lax.div(t, W - 1)
        fy = (t - y0 * (W - 1)).astype(jnp.float32) * (1.0 / (W - 1))
        ytab0[pl.ds(ch * L, L)] = y0
        ytab1[pl.ds(ch * L, L)] = jnp.minimum(y0 + 1, GH - 1)
        fytab[pl.ds(ch * L, L)] = fy
        return carry
    lax.fori_loop(0, NCHUNK, fill_y, 0)

    yiota = lax.iota(jnp.int32, L)

    def do_block(blk, carry):
        h0 = rowbase + blk * RBLK
        pltpu.sync_copy(guide_hbm.at[n, 0, pl.ds(h0, RBLK)], guidebuf)

        def do_row(rr, carry):
            h = h0 + rr
            t = h * (GW - 1)
            x0 = t // (H - 1)
            fx = (t - x0 * (H - 1)).astype(jnp.float32) * (1.0 / (H - 1))
            x1 = jnp.minimum(x0 + 1, GW - 1)
            fxv = lax.broadcast_in_dim(fx, (L,), ())
            fxc = 1.0 - fxv
            x0v = _splat_i32(x0)
            x1v = _splat_i32(x1)

            @plsc.parallel_loop(0, CD, unroll=2)
            def fold_x(cd):
                cv = _splat_i32(cd // D)
                dv = _splat_i32(cd % D)
                v0 = plsc.load_gather(gridv, [cv, dv, yiota, x0v])
                v1 = plsc.load_gather(gridv, [cv, dv, yiota, x1v])
                gxv[pl.ds(cd * GH, GH)] = v0 * fxc + v1 * fxv

            @plsc.parallel_loop(0, NCHUNK, unroll=4)
            def do_chunk(ch):
                g = guidebuf[rr, pl.ds(ch * L, L)]
                z = jnp.minimum(jnp.maximum(g * 3.5 + 3.5, 0.0), float(D - 1))
                z0 = jnp.minimum(z.astype(jnp.int32), D - 2)
                fz = z - z0.astype(jnp.float32)
                y0 = ytab0[pl.ds(ch * L, L)]
                y1 = ytab1[pl.ds(ch * L, L)]
                fy = fytab[pl.ds(ch * L, L)]
                wz0 = 1.0 - fz
                wy0 = 1.0 - fy
                w00 = wz0 * wy0
                w01 = wz0 * fy
                w10 = fz * wy0
                w11 = fz * fy
                ib0 = z0 * GH + y0
                ib1 = z0 * GH + y1
                for c in range(C):
                    o = c * (D * GH)
                    a00 = plsc.load_gather(gxv, [ib0 + o])
                    a01 = plsc.load_gather(gxv, [ib1 + o])
                    a10 = plsc.load_gather(gxv, [ib0 + (o + GH)])
                    a11 = plsc.load_gather(gxv, [ib1 + (o + GH)])
                    res = a00 * w00 + a01 * w01 + a10 * w10 + a11 * w11
                    outbuf[c, rr, pl.ds(ch * L, L)] = res
            return carry
        lax.fori_loop(0, RBLK, do_row, 0)

        for c in range(C):
            pltpu.sync_copy(outbuf.at[c], out_hbm.at[n, c, pl.ds(h0, RBLK)])
        return carry
    lax.fori_loop(0, NBLK, do_block, 0)


_SCRATCH = [
    pltpu.VMEM((C, D, GH, GW), jnp.float32),
    pltpu.VMEM((CD * GH,), jnp.float32),
    pltpu.VMEM((W,), jnp.int32),
    pltpu.VMEM((W,), jnp.int32),
    pltpu.VMEM((W,), jnp.float32),
    pltpu.VMEM((RBLK, W), jnp.float32),
    pltpu.VMEM((C, RBLK, W), jnp.float32),
]

kernel = functools.partial(
    pl.kernel,
    out_type=jax.ShapeDtypeStruct((N, C, H, W), jnp.float32),
    mesh=plsc.VectorSubcoreMesh(core_axis_name="c", subcore_axis_name="s"),
    scratch_types=_SCRATCH,
    compiler_params=pltpu.CompilerParams(needs_layout_passes=False,
                                         use_tc_tiling_on_sc=False),
)(_sc_body)


# speedup vs baseline: 1.2883x; 1.2883x over previous
import functools

import jax
import jax.numpy as jnp
from jax import lax
from jax.experimental import pallas as pl
from jax.experimental.pallas import tpu as pltpu
from jax.experimental.pallas import tpu_sc as plsc

N, C, D, GH, GW = 8, 12, 8, 16, 16
H = W = 512
L = 16
NCORES, NSUB = 2, 16
NWORK = NCORES * NSUB
ROWS_PER_W = (N * H) // NWORK
RBLK = 4
NBLK = ROWS_PER_W // RBLK
NCHUNK = W // L
CD = C * D


def _splat_i32(s):
    return lax.broadcast_in_dim(jnp.int32(s) if isinstance(s, int) else s,
                                (L,), ())


def _sc_body(grid_hbm, guide_hbm, out_hbm,
             gridv, gxv, ytab0, ytab1, fytab, guidebuf, outbuf):
    wid = lax.axis_index("s") * NCORES + lax.axis_index("c")
    n = wid // (NWORK // N)
    rowbase = (wid % (NWORK // N)) * ROWS_PER_W

    pltpu.sync_copy(grid_hbm.at[n], gridv)

    def fill_y(ch, carry):
        wv = lax.iota(jnp.int32, L) + _splat_i32(ch * L)
        t = wv * (GH - 1)
        y0 = lax.div(t, W - 1)
        fy = (t - y0 * (W - 1)).astype(jnp.float32) * (1.0 / (W - 1))
        ytab0[pl.ds(ch * L, L)] = y0
        ytab1[pl.ds(ch * L, L)] = jnp.minimum(y0 + 1, GH - 1)
        fytab[pl.ds(ch * L, L)] = fy
        return carry
    lax.fori_loop(0, NCHUNK, fill_y, 0)

    yiota = lax.iota(jnp.int32, L)

    def do_block(blk, carry):
        h0 = rowbase + blk * RBLK
        pltpu.sync_copy(guide_hbm.at[n, 0, pl.ds(h0, RBLK)], guidebuf)

        def do_row(rr, carry):
            h = h0 + rr
            t = h * (GW - 1)
            x0 = t // (H - 1)
            fx = (t - x0 * (H - 1)).astype(jnp.float32) * (1.0 / (H - 1))
            x1 = jnp.minimum(x0 + 1, GW - 1)
            fxv = lax.broadcast_in_dim(fx, (L,), ())
            fxc = 1.0 - fxv
            x0v = _splat_i32(x0)
            x1v = _splat_i32(x1)

            @plsc.parallel_loop(0, CD, unroll=2)
            def fold_x(cd):
                cv = _splat_i32(cd // D)
                dv = _splat_i32(cd % D)
                v0 = plsc.load_gather(gridv, [cv, dv, yiota, x0v])
                v1 = plsc.load_gather(gridv, [cv, dv, yiota, x1v])
                gxv[pl.ds(cd * GH, GH)] = v0 * fxc + v1 * fxv

            @plsc.parallel_loop(0, NCHUNK, unroll=2)
            def do_chunk(ch):
                g = guidebuf[rr, pl.ds(ch * L, L)]
                z = jnp.minimum(jnp.maximum(g * 3.5 + 3.5, 0.0), float(D - 1))
                z0 = jnp.minimum(z.astype(jnp.int32), D - 2)
                fz = z - z0.astype(jnp.float32)
                y0 = ytab0[pl.ds(ch * L, L)]
                y1 = ytab1[pl.ds(ch * L, L)]
                fy = fytab[pl.ds(ch * L, L)]
                wz0 = 1.0 - fz
                wy0 = 1.0 - fy
                w00 = wz0 * wy0
                w01 = wz0 * fy
                w10 = fz * wy0
                w11 = fz * fy
                ib0 = z0 * GH + y0
                ib1 = z0 * GH + y1
                for c in range(C):
                    o = c * (D * GH)
                    a00 = plsc.load_gather(gxv, [ib0 + o])
                    a01 = plsc.load_gather(gxv, [ib1 + o])
                    a10 = plsc.load_gather(gxv, [ib0 + (o + GH)])
                    a11 = plsc.load_gather(gxv, [ib1 + (o + GH)])
                    res = a00 * w00 + a01 * w01 + a10 * w10 + a11 * w11
                    outbuf[c, rr, pl.ds(ch * L, L)] = res
            return carry
        lax.fori_loop(0, RBLK, do_row, 0)

        pltpu.sync_copy(outbuf, out_hbm.at[n, :, pl.ds(h0, RBLK)])
        return carry
    lax.fori_loop(0, NBLK, do_block, 0)


_SCRATCH = [
    pltpu.VMEM((C, D, GH, GW), jnp.float32),
    pltpu.VMEM((CD * GH,), jnp.float32),
    pltpu.VMEM((W,), jnp.int32),
    pltpu.VMEM((W,), jnp.int32),
    pltpu.VMEM((W,), jnp.float32),
    pltpu.VMEM((RBLK, W), jnp.float32),
    pltpu.VMEM((C, RBLK, W), jnp.float32),
]

kernel = functools.partial(
    pl.kernel,
    out_type=jax.ShapeDtypeStruct((N, C, H, W), jnp.float32),
    mesh=plsc.VectorSubcoreMesh(core_axis_name="c", subcore_axis_name="s"),
    scratch_types=_SCRATCH,
    compiler_params=pltpu.CompilerParams(needs_layout_passes=False,
                                         use_tc_tiling_on_sc=False),
)(_sc_body)


# speedup vs baseline: 1.8081x; 1.4035x over previous
import functools

import jax
import jax.numpy as jnp
from jax import lax
from jax.experimental import pallas as pl
from jax.experimental.pallas import tpu as pltpu
from jax.experimental.pallas import tpu_sc as plsc

N, C, D, GH, GW = 8, 12, 8, 16, 16
H = W = 512
L = 16
NCORES, NSUB = 2, 16
NWORK = NCORES * NSUB
ROWS_PER_W = (N * H) // NWORK
RBLK = 4
NBLK = ROWS_PER_W // RBLK
NCHUNK = W // L
CD = C * D
GXV_PAD = CD * GH + 24


def _splat_i32(s):
    return lax.broadcast_in_dim(jnp.int32(s) if isinstance(s, int) else s,
                                (L,), ())


def _sc_body(grid_hbm, guide_hbm, out_hbm,
             gridv, gxt, gxv, ytab0, fytab, guidebuf, outbuf):
    wid = lax.axis_index("s") * NCORES + lax.axis_index("c")
    n = wid // (NWORK // N)
    rowbase = (wid % (NWORK // N)) * ROWS_PER_W

    pltpu.sync_copy(grid_hbm.at[n], gridv)

    def fill_y(ch, carry):
        wv = lax.iota(jnp.int32, L) + _splat_i32(ch * L)
        t = wv * (GH - 1)
        y0 = lax.div(t, W - 1)
        fy = (t - y0 * (W - 1)).astype(jnp.float32) * (1.0 / (W - 1))
        ytab0[pl.ds(ch * L, L)] = y0
        fytab[pl.ds(ch * L, L)] = fy
        return carry
    lax.fori_loop(0, NCHUNK, fill_y, 0)

    yiota = lax.iota(jnp.int32, L)

    def build_t(cd, carry):
        cv = _splat_i32(lax.div(cd, D))
        dv = _splat_i32(lax.rem(cd, D))
        for x in range(GW):
            col = plsc.load_gather(gridv, [cv, dv, yiota, _splat_i32(x)])
            gxt[pl.ds(cd * (GW * GH) + x * GH, GH)] = col
        return carry
    lax.fori_loop(0, CD, build_t, 0)

    zpad = jnp.zeros((L,), jnp.float32)
    gxv[pl.ds(CD * GH, L)] = zpad
    gxv[pl.ds(GXV_PAD - L, L)] = zpad

    def do_block(blk, carry):
        h0 = rowbase + blk * RBLK
        pltpu.sync_copy(guide_hbm.at[n, 0, pl.ds(h0, RBLK)], guidebuf)

        def do_row(rr, carry):
            h = h0 + rr
            t = h * (GW - 1)
            x0 = t // (H - 1)
            fx = (t - x0 * (H - 1)).astype(jnp.float32) * (1.0 / (H - 1))
            x1 = jnp.minimum(x0 + 1, GW - 1)
            fxv = lax.broadcast_in_dim(fx, (L,), ())
            fxc = 1.0 - fxv
            xoff = x0 * GH
            dx = (x1 - x0) * GH

            @plsc.parallel_loop(0, CD, unroll=2)
            def fold_x(cd):
                b0 = cd * (GW * GH) + xoff
                v0 = gxt[pl.ds(b0, GH)]
                v1 = gxt[pl.ds(b0 + dx, GH)]
                gxv[pl.ds(cd * GH, GH)] = v0 * fxc + v1 * fxv

            @plsc.parallel_loop(0, NCHUNK, unroll=2)
            def do_chunk(ch):
                g = guidebuf[rr, pl.ds(ch * L, L)]
                z = jnp.minimum(jnp.maximum(g * 3.5 + 3.5, 0.0), float(D - 1))
                z0 = jnp.minimum(z.astype(jnp.int32), D - 2)
                fz = z - z0.astype(jnp.float32)
                y0 = ytab0[pl.ds(ch * L, L)]
                fy = fytab[pl.ds(ch * L, L)]
                wz0 = 1.0 - fz
                wy0 = 1.0 - fy
                w00 = wz0 * wy0
                w01 = wz0 * fy
                w10 = fz * wy0
                w11 = fz * fy
                ib0 = z0 * GH + y0
                ib1 = ib0 + 1
                for c in range(C):
                    o = c * (D * GH)
                    a00 = plsc.load_gather(gxv.at[pl.ds(o, 128)], [ib0])
                    a01 = plsc.load_gather(gxv.at[pl.ds(o, 128)], [ib1])
                    a10 = plsc.load_gather(gxv.at[pl.ds(o + GH, 128)], [ib0])
                    a11 = plsc.load_gather(gxv.at[pl.ds(o + GH, 128)], [ib1])
                    res = a00 * w00 + a01 * w01 + a10 * w10 + a11 * w11
                    outbuf[c, rr, pl.ds(ch * L, L)] = res
            return carry
        lax.fori_loop(0, RBLK, do_row, 0)

        pltpu.sync_copy(outbuf, out_hbm.at[n, :, pl.ds(h0, RBLK)])
        return carry
    lax.fori_loop(0, NBLK, do_block, 0)


_SCRATCH = [
    pltpu.VMEM((C, D, GH, GW), jnp.float32),
    pltpu.VMEM((CD * GW * GH,), jnp.float32),
    pltpu.VMEM((GXV_PAD,), jnp.float32),
    pltpu.VMEM((W,), jnp.int32),
    pltpu.VMEM((W,), jnp.float32),
    pltpu.VMEM((RBLK, W), jnp.float32),
    pltpu.VMEM((C, RBLK, W), jnp.float32),
]

kernel = functools.partial(
    pl.kernel,
    out_type=jax.ShapeDtypeStruct((N, C, H, W), jnp.float32),
    mesh=plsc.VectorSubcoreMesh(core_axis_name="c", subcore_axis_name="s"),
    scratch_types=_SCRATCH,
    compiler_params=pltpu.CompilerParams(needs_layout_passes=False,
                                         use_tc_tiling_on_sc=False),
)(_sc_body)
